# baseline (speedup 1.0000x reference)
import functools

import numpy as np
import jax
import jax.numpy as jnp
from jax import lax
from jax.experimental import pallas as pl
from jax.experimental.pallas import tpu as pltpu
from jax.experimental.pallas import tpu_sc as plsc

_B = 128
_V = 100000
_CH = 10000
_NCH = _V // _CH
_NV = _CH // 16
_NC = 2
_NS = 16
_NW = _NC * _NS
_RPW = _B // _NW
_OUTP = 16
_EPS = 1e-10
_IBIG = np.int32(2**31 - 1)

_consts = []


def _threefry2x32_np(k1, k2, x0, x1):
    r1 = (13, 15, 26, 6)
    r2 = (17, 29, 16, 24)

    def rl(x, d):
        return (x << np.uint32(d)) | (x >> np.uint32(32 - d))

    def rounds(x0, x1, rots):
        for r in rots:
            x0 = x0 + x1
            x1 = rl(x1, r)
            x1 = x1 ^ x0
        return x0, x1

    ks0 = np.uint32(k1)
    ks1 = np.uint32(k2)
    ks2 = np.uint32(0x1BD11BDA) ^ ks0 ^ ks1
    x0 = x0 + ks0
    x1 = x1 + ks1
    x0, x1 = rounds(x0, x1, r1)
    x0 = x0 + ks1
    x1 = x1 + (ks2 + np.uint32(1))
    x0, x1 = rounds(x0, x1, r2)
    x0 = x0 + ks2
    x1 = x1 + (ks0 + np.uint32(2))
    x0, x1 = rounds(x0, x1, r1)
    x0 = x0 + ks0
    x1 = x1 + (ks1 + np.uint32(3))
    x0, x1 = rounds(x0, x1, r2)
    x0 = x0 + ks1
    x1 = x1 + (ks2 + np.uint32(4))
    x0, x1 = rounds(x0, x1, r1)
    x0 = x0 + ks2
    x1 = x1 + (ks0 + np.uint32(5))
    return x0, x1


def _noise_recip():
    if not _consts:
        o0, o1 = _threefry2x32_np(np.uint32(0), np.uint32(0),
                                  np.uint32([0]), np.uint32([12345]))
        k1, k2 = o0[0], o1[0]
        idx = np.arange(_B * _V, dtype=np.uint32)
        b0, b1 = _threefry2x32_np(k1, k2, np.zeros_like(idx), idx)
        bits = b0 ^ b1
        fb = (bits >> np.uint32(9)) | np.uint32(0x3F800000)
        u = fb.view(np.float32) - np.float32(1.0)
        n = -np.log1p(-u)
        r = (1.0 / (n.astype(np.float64) + _EPS)).astype(np.float32)
        _consts.append(r)
    return _consts[0]


def _sampler_sc_body(logits_hbm, recip_hbm, temps_hbm, out_hbm,
                     lbuf, rbuf, qbuf, tbuf, obuf):
    wid = lax.axis_index("s") * _NC + lax.axis_index("c")
    pltpu.sync_copy(temps_hbm, tbuf)
    lane = lax.iota(jnp.int32, 16)
    neg_inf = jnp.float32(-jnp.inf)
    tokens = jnp.zeros((16,), jnp.int32)

    for r in range(_RPW):
        row = wid * _RPW + r
        t = plsc.load_gather(tbuf, [jnp.full((16,), row, jnp.int32)])

        def chunk_body(c, carry):
            m_run, sbest, sidx, gbest, gidx = carry
            base = c * _CH
            flat = row * _V + base
            pltpu.sync_copy(logits_hbm.at[pl.ds(flat, _CH)], lbuf)
            pltpu.sync_copy(recip_hbm.at[pl.ds(flat, _CH)], rbuf)

            def sweep1(i, carry1):
                mv, gb, gi = carry1
                sl = pl.ds(i * 16, 16)
                l = lbuf[sl]
                q = l / t
                qbuf[sl] = q
                vidx = (base + i * 16) + lane
                upd = l > gb
                return (jnp.maximum(mv, q),
                        jnp.where(upd, l, gb),
                        jnp.where(upd, vidx, gi))

            mv0 = jnp.full((16,), neg_inf, jnp.float32)
            mv, gbest, gidx = lax.fori_loop(0, _NV, sweep1,
                                            (mv0, gbest, gidx))
            m_new = jnp.maximum(m_run, jnp.max(mv))
            scale = jnp.exp(jnp.full((16,), m_run - m_new, jnp.float32))
            sbest = sbest * scale
            m_vec = jnp.full((16,), m_new, jnp.float32)

            def sweep2(i, carry2):
                sb, si = carry2
                sl = pl.ds(i * 16, 16)
                s = jnp.exp(qbuf[sl] - m_vec) * rbuf[sl]
                vidx = (base + i * 16) + lane
                upd = s > sb
                return (jnp.where(upd, s, sb), jnp.where(upd, vidx, si))

            sbest, sidx = lax.fori_loop(0, _NV, sweep2, (sbest, sidx))
            return (m_new, sbest, sidx, gbest, gidx)

        init = (neg_inf,
                jnp.zeros((16,), jnp.float32), jnp.zeros((16,), jnp.int32),
                jnp.full((16,), neg_inf, jnp.float32),
                jnp.zeros((16,), jnp.int32))
        _, sbest, sidx, gbest, gidx = lax.fori_loop(0, _NCH, chunk_body, init)

        ibig = jnp.full((16,), _IBIG, jnp.int32)
        gmax = jnp.full((16,), jnp.max(gbest), jnp.float32)
        gtok = jnp.min(jnp.where(gbest == gmax, gidx, ibig))
        smax = jnp.full((16,), jnp.max(sbest), jnp.float32)
        stok = jnp.min(jnp.where(sbest == smax, sidx, ibig))
        tok = jnp.where(t == jnp.float32(0.0),
                        jnp.full((16,), gtok, jnp.int32),
                        jnp.full((16,), stok, jnp.int32))
        tokens = jnp.where(lane == r, tok, tokens)

    obuf[...] = tokens
    pltpu.sync_copy(obuf, out_hbm.at[pl.ds(wid * _OUTP, _OUTP)])


_sampler_cache = []


def _sampler_sc():
    if not _sampler_cache:
        _sampler_cache.append(pl.kernel(
            _sampler_sc_body,
            out_type=jax.ShapeDtypeStruct((_NW * _OUTP,), jnp.int32),
            mesh=plsc.VectorSubcoreMesh(core_axis_name="c",
                                        subcore_axis_name="s",
                                        num_cores=_NC, num_subcores=_NS),
            scratch_types=[
                pltpu.VMEM((_CH,), jnp.float32),
                pltpu.VMEM((_CH,), jnp.float32),
                pltpu.VMEM((_CH,), jnp.float32),
                pltpu.VMEM((_B,), jnp.float32),
                pltpu.VMEM((_OUTP,), jnp.int32),
            ],
            compiler_params=pltpu.CompilerParams(needs_layout_passes=False),
        ))
    return _sampler_cache[0]


def kernel(logits, temperatures):
    recip = jnp.asarray(_noise_recip())
    flat = _sampler_sc()(logits.reshape(_B * _V), recip, temperatures)
    return flat.reshape(_NW, _OUTP)[:, :_RPW].reshape(_B)


# speedup vs baseline: 1.0698x; 1.0698x over previous
import functools

import numpy as np
import jax
import jax.numpy as jnp
from jax import lax
from jax.experimental import pallas as pl
from jax.experimental.pallas import tpu as pltpu
from jax.experimental.pallas import tpu_sc as plsc

_B = 128
_V = 100000
_CH = 10000
_NCH = _V // _CH
_NV = _CH // 16
_NC = 2
_NS = 16
_NW = _NC * _NS
_RPW = _B // _NW
_OUTP = 16
_EPS = 1e-10
_IBIG = np.int32(2**31 - 1)

_consts = []


def _threefry2x32_np(k1, k2, x0, x1):
    r1 = (13, 15, 26, 6)
    r2 = (17, 29, 16, 24)

    def rl(x, d):
        return (x << np.uint32(d)) | (x >> np.uint32(32 - d))

    def rounds(x0, x1, rots):
        for r in rots:
            x0 = x0 + x1
            x1 = rl(x1, r)
            x1 = x1 ^ x0
        return x0, x1

    ks0 = np.uint32(k1)
    ks1 = np.uint32(k2)
    ks2 = np.uint32(0x1BD11BDA) ^ ks0 ^ ks1
    x0 = x0 + ks0
    x1 = x1 + ks1
    x0, x1 = rounds(x0, x1, r1)
    x0 = x0 + ks1
    x1 = x1 + (ks2 + np.uint32(1))
    x0, x1 = rounds(x0, x1, r2)
    x0 = x0 + ks2
    x1 = x1 + (ks0 + np.uint32(2))
    x0, x1 = rounds(x0, x1, r1)
    x0 = x0 + ks0
    x1 = x1 + (ks1 + np.uint32(3))
    x0, x1 = rounds(x0, x1, r2)
    x0 = x0 + ks1
    x1 = x1 + (ks2 + np.uint32(4))
    x0, x1 = rounds(x0, x1, r1)
    x0 = x0 + ks2
    x1 = x1 + (ks0 + np.uint32(5))
    return x0, x1


def _noise_recip():
    if not _consts:
        o0, o1 = _threefry2x32_np(np.uint32(0), np.uint32(0),
                                  np.uint32([0]), np.uint32([12345]))
        k1, k2 = o0[0], o1[0]
        idx = np.arange(_B * _V, dtype=np.uint32)
        b0, b1 = _threefry2x32_np(k1, k2, np.zeros_like(idx), idx)
        bits = b0 ^ b1
        fb = (bits >> np.uint32(9)) | np.uint32(0x3F800000)
        u = fb.view(np.float32) - np.float32(1.0)
        n = -np.log1p(-u)
        r = (1.0 / (n.astype(np.float64) + _EPS)).astype(np.float32)
        _consts.append(r)
    return _consts[0]


def _sampler_sc_body(logits_hbm, recip_hbm, temps_hbm, out_hbm,
                     lbuf, rbuf, qbuf, tbuf, obuf):
    wid = lax.axis_index("s") * _NC + lax.axis_index("c")
    pltpu.sync_copy(temps_hbm, tbuf)
    lane = lax.iota(jnp.int32, 16)
    neg_inf = jnp.float32(-jnp.inf)
    tokens = jnp.zeros((16,), jnp.int32)

    for r in range(_RPW):
        row = wid * _RPW + r
        t = plsc.load_gather(tbuf, [jnp.full((16,), row, jnp.int32)])

        def chunk_body(c, carry):
            m_run, sbest, sidx, gbest, gidx = carry
            base = c * _CH
            flat = row * _V + base
            pltpu.sync_copy(logits_hbm.at[pl.ds(flat, _CH)], lbuf)
            pltpu.sync_copy(recip_hbm.at[pl.ds(flat, _CH)], rbuf)

            def sweep1(i, carry1):
                mv, gb, gi = carry1
                sl = pl.ds(i * 16, 16)
                l = lbuf[sl]
                q = l / t
                qbuf[sl] = q
                vidx = (base + i * 16) + lane
                upd = l > gb
                return (jnp.maximum(mv, q),
                        jnp.where(upd, l, gb),
                        jnp.where(upd, vidx, gi))

            mv0 = jnp.full((16,), neg_inf, jnp.float32)
            mv, gbest, gidx = lax.fori_loop(0, _NV, sweep1,
                                            (mv0, gbest, gidx), unroll=5)
            m_new = jnp.maximum(m_run, jnp.max(mv))
            scale = jnp.exp(jnp.full((16,), m_run - m_new, jnp.float32))
            sbest = sbest * scale
            m_vec = jnp.full((16,), m_new, jnp.float32)

            def sweep2(i, carry2):
                sb, si = carry2
                sl = pl.ds(i * 16, 16)
                s = jnp.exp(qbuf[sl] - m_vec) * rbuf[sl]
                vidx = (base + i * 16) + lane
                upd = s > sb
                return (jnp.where(upd, s, sb), jnp.where(upd, vidx, si))

            sbest, sidx = lax.fori_loop(0, _NV, sweep2, (sbest, sidx),
                                        unroll=5)
            return (m_new, sbest, sidx, gbest, gidx)

        init = (neg_inf,
                jnp.zeros((16,), jnp.float32), jnp.zeros((16,), jnp.int32),
                jnp.full((16,), neg_inf, jnp.float32),
                jnp.zeros((16,), jnp.int32))
        _, sbest, sidx, gbest, gidx = lax.fori_loop(0, _NCH, chunk_body, init)

        ibig = jnp.full((16,), _IBIG, jnp.int32)
        gmax = jnp.full((16,), jnp.max(gbest), jnp.float32)
        gtok = jnp.min(jnp.where(gbest == gmax, gidx, ibig))
        smax = jnp.full((16,), jnp.max(sbest), jnp.float32)
        stok = jnp.min(jnp.where(sbest == smax, sidx, ibig))
        tok = jnp.where(t == jnp.float32(0.0),
                        jnp.full((16,), gtok, jnp.int32),
                        jnp.full((16,), stok, jnp.int32))
        tokens = jnp.where(lane == r, tok, tokens)

    obuf[...] = tokens
    pltpu.sync_copy(obuf, out_hbm.at[pl.ds(wid * _OUTP, _OUTP)])


_sampler_cache = []


def _sampler_sc():
    if not _sampler_cache:
        _sampler_cache.append(pl.kernel(
            _sampler_sc_body,
            out_type=jax.ShapeDtypeStruct((_NW * _OUTP,), jnp.int32),
            mesh=plsc.VectorSubcoreMesh(core_axis_name="c",
                                        subcore_axis_name="s",
                                        num_cores=_NC, num_subcores=_NS),
            scratch_types=[
                pltpu.VMEM((_CH,), jnp.float32),
                pltpu.VMEM((_CH,), jnp.float32),
                pltpu.VMEM((_CH,), jnp.float32),
                pltpu.VMEM((_B,), jnp.float32),
                pltpu.VMEM((_OUTP,), jnp.int32),
            ],
            compiler_params=pltpu.CompilerParams(needs_layout_passes=False),
        ))
    return _sampler_cache[0]


def kernel(logits, temperatures):
    recip = jnp.asarray(_noise_recip())
    flat = _sampler_sc()(logits.reshape(_B * _V), recip, temperatures)
    return flat.reshape(_NW, _OUTP)[:, :_RPW].reshape(_B)
